# Initial kernel scaffold; baseline (speedup 1.0000x reference)
import jax
import jax.numpy as jnp
from jax.experimental import pallas as pl


def kernel(inputs, targets, bsce_weight):
    raise NotImplementedError("write your pallas kernel here")


# speedup vs baseline: 12.2643x; 12.2643x over previous
import functools

import jax
import jax.numpy as jnp
from jax.experimental import pallas as pl

SCALE = 16.0
HCM_N = 30
FACTOR = 0.6

_NEG_INF = float("-inf")


def _stats(v, onehot_f):
    m = jnp.max(v, axis=1, keepdims=True)
    ex = jnp.exp(v - m)
    e = jnp.sum(ex, axis=1, keepdims=True)
    p = ex / e
    loge = jnp.log(e)
    logp = (v - m) - loge
    lp = jnp.log(p + 1e-9)
    s_row = jnp.sum(p * logp, axis=1)
    pick = jnp.sum(v * onehot_f, axis=1)
    lse = (m + loge)[:, 0]
    return lse, p, lp, s_row, pick


def _block_body(x_ref, t_ref, w_ref, out_ref, *, bb, n, c):
    a0 = x_ref[0]
    t0 = t_ref[0, 0]
    logw = jnp.log(w_ref[...])
    iota = jax.lax.broadcasted_iota(jnp.int32, (bb, n), 1)
    onehot = iota == t0[:, None]
    onehot_f = onehot.astype(jnp.float32)

    cs = jnp.where(onehot, 999999.0, a0)
    mask = jnp.zeros((bb, n), jnp.float32)
    avail = cs
    for _ in range(HCM_N):
        mval = jnp.max(avail, axis=1, keepdims=True)
        sel = avail == mval
        mask = jnp.where(sel, 1.0, mask)
        avail = jnp.where(sel, _NEG_INF, avail)

    t_bal = jnp.float32(0.0)
    t_hcm = jnp.float32(0.0)
    ce_bal = jnp.float32(0.0)
    ce_hcm = jnp.float32(0.0)
    for group in range(2):
        ps, lps, s_tot = [], [], jnp.float32(0.0)
        ce = jnp.float32(0.0)
        for i in range(c):
            logits = x_ref[i] * SCALE + logw
            if group == 1:
                logits = logits * mask
            lse, p, lp, s_row, pick = _stats(logits, onehot_f)
            ps.append(p)
            lps.append(lp)
            s_tot += jnp.sum(s_row)
            ce += jnp.sum(lse - pick)
        x_cross = jnp.float32(0.0)
        for i in range(c):
            for j in range(c):
                if i == j:
                    continue
                x_cross += jnp.sum(ps[j] * lps[i])
        t_grp = (c - 1) * s_tot - x_cross
        if group == 0:
            t_bal, ce_bal = t_grp, ce
        else:
            t_hcm, ce_hcm = t_grp, ce

    lane = jax.lax.broadcasted_iota(jnp.int32, (1, 128), 1)
    out = jnp.where(lane == 0, t_bal, 0.0)
    out = jnp.where(lane == 1, t_hcm, out)
    out = jnp.where(lane == 2, ce_bal, out)
    out = jnp.where(lane == 3, ce_hcm, out)
    out_ref[...] = out[None].astype(jnp.float32)


def kernel(inputs, targets, bsce_weight):
    c, b, n = inputs.shape
    bb = 512
    grid = b // bb
    t3 = targets[0].reshape(grid, 1, bb)
    w2 = bsce_weight.reshape(1, n)

    body = functools.partial(_block_body, bb=bb, n=n, c=c)
    partials = pl.pallas_call(
        body,
        grid=(grid,),
        in_specs=[
            pl.BlockSpec((c, bb, n), lambda g: (0, g, 0)),
            pl.BlockSpec((1, 1, bb), lambda g: (g, 0, 0)),
            pl.BlockSpec((1, n), lambda g: (0, 0)),
        ],
        out_specs=pl.BlockSpec((1, 1, 128), lambda g: (g, 0, 0)),
        out_shape=jax.ShapeDtypeStruct((grid, 1, 128), jnp.float32),
    )(inputs, t3, w2)

    s = jnp.sum(partials, axis=(0, 1))
    t_bal, t_hcm, ce_bal, ce_hcm = s[0], s[1], s[2], s[3]
    loss = (FACTOR * (t_bal + t_hcm) / (c - 1) + ce_bal + ce_hcm) / b
    return loss.astype(jnp.float32)


# speedup vs baseline: 14.5248x; 1.1843x over previous
import functools

import jax
import jax.numpy as jnp
from jax.experimental import pallas as pl

SCALE = 16.0
HCM_N = 30
FACTOR = 0.6

_NEG_INF = float("-inf")


def _block_body(x_ref, t_ref, w_ref, out_ref, *, bb, n, c):
    a0 = x_ref[0]
    t0 = t_ref[0, 0]
    logw = jnp.log(w_ref[...])
    iota = jax.lax.broadcasted_iota(jnp.int32, (bb, n), 1)
    onehot_f = (iota == t0[:, None]).astype(jnp.float32)

    cs = a0 + onehot_f * 1e9
    avail = cs
    for _ in range(HCM_N):
        mval = jnp.max(avail, axis=1, keepdims=True)
        avail = jnp.where(avail == mval, _NEG_INF, avail)
    mask = jnp.where(avail == _NEG_INF, 1.0, 0.0)

    ls = [x_ref[i] * SCALE + logw for i in range(c)]

    t_terms = []
    ce_terms = []
    for group in range(2):
        ps, lps = [], []
        s_tot = None
        ce = None
        for i in range(c):
            if group == 0:
                v = ls[i]
            else:
                v = ls[i] * mask
            m = jnp.max(v, axis=1, keepdims=True)
            ex = jnp.exp(v - m)
            e = jnp.sum(ex, axis=1, keepdims=True)
            loge = jnp.log(e)
            p = ex * (1.0 / e)
            logp = (v - m) - loge
            lp = jnp.log(p + 1e-9)
            ps.append(p)
            lps.append(lp)
            s_i = jnp.sum(p * logp)
            ce_i = jnp.sum(m + loge) - jnp.sum(v * onehot_f)
            s_tot = s_i if s_tot is None else s_tot + s_i
            ce = ce_i if ce is None else ce + ce_i
        x_cross = None
        for i in range(c):
            for j in range(c):
                if i == j:
                    continue
                d = jnp.sum(ps[j] * lps[i])
                x_cross = d if x_cross is None else x_cross + d
        t_terms.append((c - 1) * s_tot - x_cross)
        ce_terms.append(ce)

    lane = jax.lax.broadcasted_iota(jnp.int32, (1, 128), 1)
    out = jnp.where(lane == 0, t_terms[0], 0.0)
    out = jnp.where(lane == 1, t_terms[1], out)
    out = jnp.where(lane == 2, ce_terms[0], out)
    out = jnp.where(lane == 3, ce_terms[1], out)
    out_ref[...] = out[None].astype(jnp.float32)


def kernel(inputs, targets, bsce_weight):
    c, b, n = inputs.shape
    bb = 1024
    grid = b // bb
    t3 = targets[0].reshape(grid, 1, bb)
    w2 = bsce_weight.reshape(1, n)

    body = functools.partial(_block_body, bb=bb, n=n, c=c)
    partials = pl.pallas_call(
        body,
        grid=(grid,),
        in_specs=[
            pl.BlockSpec((c, bb, n), lambda g: (0, g, 0)),
            pl.BlockSpec((1, 1, bb), lambda g: (g, 0, 0)),
            pl.BlockSpec((1, n), lambda g: (0, 0)),
        ],
        out_specs=pl.BlockSpec((1, 1, 128), lambda g: (g, 0, 0)),
        out_shape=jax.ShapeDtypeStruct((grid, 1, 128), jnp.float32),
    )(inputs, t3, w2)

    s = jnp.sum(partials, axis=(0, 1))
    t_bal, t_hcm, ce_bal, ce_hcm = s[0], s[1], s[2], s[3]
    loss = (FACTOR * (t_bal + t_hcm) / (c - 1) + ce_bal + ce_hcm) / b
    return loss.astype(jnp.float32)


# speedup vs baseline: 14.7477x; 1.0153x over previous
import functools

import jax
import jax.numpy as jnp
from jax.experimental import pallas as pl

SCALE = 16.0
HCM_N = 30
FACTOR = 0.6

_NEG_INF = float("-inf")


def _block_body(x_ref, t_ref, w_ref, out_ref, *, bb, n, c):
    a0 = x_ref[0]
    t0 = t_ref[0, 0]
    logw = jnp.log(w_ref[...])
    iota = jax.lax.broadcasted_iota(jnp.int32, (bb, n), 1)
    onehot_f = (iota == t0[:, None]).astype(jnp.float32)

    avail0 = jnp.where(onehot_f > 0, _NEG_INF, a0)
    nchain = 4
    rows = bb // nchain
    chains = [avail0[k * rows:(k + 1) * rows] for k in range(nchain)]
    for _ in range(HCM_N - 1):
        mvals = [jnp.max(av, axis=1, keepdims=True) for av in chains]
        chains = [jnp.where(av == mv, _NEG_INF, av)
                  for av, mv in zip(chains, mvals)]
    avail = jnp.concatenate(chains, axis=0)
    mask = jnp.where(avail == _NEG_INF, 1.0, 0.0)

    ls = [x_ref[i] * SCALE + logw for i in range(c)]

    t_terms = []
    ce_terms = []
    for group in range(2):
        ps, lps = [], []
        s_tot = None
        ce = None
        for i in range(c):
            if group == 0:
                v = ls[i]
            else:
                v = ls[i] * mask
            m = jnp.max(v, axis=1, keepdims=True)
            ex = jnp.exp(v - m)
            e = jnp.sum(ex, axis=1, keepdims=True)
            loge = jnp.log(e)
            p = ex * (1.0 / e)
            logp = (v - m) - loge
            lp = jnp.log(p + 1e-9)
            ps.append(p)
            lps.append(lp)
            s_i = jnp.sum(p * logp)
            ce_i = jnp.sum(m + loge) - jnp.sum(v * onehot_f)
            s_tot = s_i if s_tot is None else s_tot + s_i
            ce = ce_i if ce is None else ce + ce_i
        x_cross = None
        for i in range(c):
            for j in range(c):
                if i == j:
                    continue
                d = jnp.sum(ps[j] * lps[i])
                x_cross = d if x_cross is None else x_cross + d
        t_terms.append((c - 1) * s_tot - x_cross)
        ce_terms.append(ce)

    lane = jax.lax.broadcasted_iota(jnp.int32, (1, 128), 1)
    out = jnp.where(lane == 0, t_terms[0], 0.0)
    out = jnp.where(lane == 1, t_terms[1], out)
    out = jnp.where(lane == 2, ce_terms[0], out)
    out = jnp.where(lane == 3, ce_terms[1], out)
    out_ref[...] = out[None].astype(jnp.float32)


def kernel(inputs, targets, bsce_weight):
    c, b, n = inputs.shape
    bb = 1024
    grid = b // bb
    t3 = targets[0].reshape(grid, 1, bb)
    w2 = bsce_weight.reshape(1, n)

    body = functools.partial(_block_body, bb=bb, n=n, c=c)
    partials = pl.pallas_call(
        body,
        grid=(grid,),
        in_specs=[
            pl.BlockSpec((c, bb, n), lambda g: (0, g, 0)),
            pl.BlockSpec((1, 1, bb), lambda g: (g, 0, 0)),
            pl.BlockSpec((1, n), lambda g: (0, 0)),
        ],
        out_specs=pl.BlockSpec((1, 1, 128), lambda g: (g, 0, 0)),
        out_shape=jax.ShapeDtypeStruct((grid, 1, 128), jnp.float32),
    )(inputs, t3, w2)

    s = jnp.sum(partials, axis=(0, 1))
    t_bal, t_hcm, ce_bal, ce_hcm = s[0], s[1], s[2], s[3]
    loss = (FACTOR * (t_bal + t_hcm) / (c - 1) + ce_bal + ce_hcm) / b
    return loss.astype(jnp.float32)


# speedup vs baseline: 15.6730x; 1.0627x over previous
import functools

import jax
import jax.numpy as jnp
from jax.experimental import pallas as pl

SCALE = 16.0
HCM_N = 30
FACTOR = 0.6

_NEG_INF = float("-inf")


def _block_body(x_ref, t_ref, w_ref, out_ref, *, bb, n, c):
    a0 = x_ref[0]
    t0 = t_ref[0, 0]
    logw = jnp.log(w_ref[...])
    iota = jax.lax.broadcasted_iota(jnp.int32, (bb, n), 1)
    onehot_f = (iota == t0[:, None]).astype(jnp.float32)

    avail0 = jnp.where(onehot_f > 0, _NEG_INF, a0)
    mth = jnp.max(avail0, axis=1, keepdims=True)
    for _ in range(HCM_N - 2):
        cand = jnp.where(avail0 < mth, avail0, _NEG_INF)
        mth = jnp.max(cand, axis=1, keepdims=True)
    mask = jnp.maximum(onehot_f, (avail0 >= mth).astype(jnp.float32))

    ls = [x_ref[i] * SCALE + logw for i in range(c)]
    lsum = ls[0] + ls[1] + ls[2]
    pick = jnp.sum(lsum * onehot_f)

    ones_mat = jnp.ones((n, 128), jnp.float32)
    dot_dims = (((1,), (0,)), ((), ()))

    def rowsum(x):
        full = jax.lax.dot_general(x, ones_mat, dot_dims,
                                   preferred_element_type=jnp.float32)
        return full[:, :1]

    lwmax = jnp.max(logw)
    amax = jnp.max(jnp.maximum(jnp.maximum(a0, x_ref[1]), x_ref[2]),
                   axis=1, keepdims=True)
    mb = amax * SCALE + lwmax

    t_terms = []
    ce_terms = []
    for group in range(2):
        p_tot = None
        lp_tot = None
        diag = None
        ent = None
        lse_sum = None
        for i in range(c):
            if group == 0:
                v = ls[i]
                m = mb
            else:
                v = ls[i] * mask
                m = jnp.max(v, axis=1, keepdims=True)
            vm = v - m
            ex = jnp.exp(vm)
            e = jnp.sum(ex, axis=1, keepdims=True)
            loge = jnp.log(e)
            p = ex * (1.0 / e)
            logp = vm - loge
            lp = jnp.log(p + 1e-9)
            e_i = p * logp
            d_i = p * lp
            l_i = jnp.sum(m + loge)
            p_tot = p if p_tot is None else p_tot + p
            lp_tot = lp if lp_tot is None else lp_tot + lp
            ent = e_i if ent is None else ent + e_i
            diag = d_i if diag is None else diag + d_i
            lse_sum = l_i if lse_sum is None else lse_sum + l_i
        x_cross = jnp.sum(p_tot * lp_tot - diag)
        s_tot = jnp.sum(ent)
        t_terms.append((c - 1) * s_tot - x_cross)
        ce_terms.append(lse_sum - pick)

    lane = jax.lax.broadcasted_iota(jnp.int32, (1, 128), 1)
    out = jnp.where(lane == 0, t_terms[0], 0.0)
    out = jnp.where(lane == 1, t_terms[1], out)
    out = jnp.where(lane == 2, ce_terms[0], out)
    out = jnp.where(lane == 3, ce_terms[1], out)
    out_ref[...] = out[None].astype(jnp.float32)


def kernel(inputs, targets, bsce_weight):
    c, b, n = inputs.shape
    bb = 1024
    grid = b // bb
    t3 = targets[0].reshape(grid, 1, bb)
    w2 = bsce_weight.reshape(1, n)

    body = functools.partial(_block_body, bb=bb, n=n, c=c)
    partials = pl.pallas_call(
        body,
        grid=(grid,),
        in_specs=[
            pl.BlockSpec((c, bb, n), lambda g: (0, g, 0)),
            pl.BlockSpec((1, 1, bb), lambda g: (g, 0, 0)),
            pl.BlockSpec((1, n), lambda g: (0, 0)),
        ],
        out_specs=pl.BlockSpec((1, 1, 128), lambda g: (g, 0, 0)),
        out_shape=jax.ShapeDtypeStruct((grid, 1, 128), jnp.float32),
    )(inputs, t3, w2)

    s = jnp.sum(partials, axis=(0, 1))
    t_bal, t_hcm, ce_bal, ce_hcm = s[0], s[1], s[2], s[3]
    loss = (FACTOR * (t_bal + t_hcm) / (c - 1) + ce_bal + ce_hcm) / b
    return loss.astype(jnp.float32)


# speedup vs baseline: 15.7854x; 1.0072x over previous
import functools

import jax
import jax.numpy as jnp
from jax.experimental import pallas as pl

SCALE = 16.0
HCM_N = 30
FACTOR = 0.6

_NEG_INF = float("-inf")


def _block_body(x_ref, t_ref, w_ref, out_ref, *, bb, n, c):
    a0 = x_ref[0]
    t0 = t_ref[0, 0]
    logw = jnp.log(w_ref[...])
    iota = jax.lax.broadcasted_iota(jnp.int32, (bb, n), 1)
    onehot_f = (iota == t0[:, None]).astype(jnp.float32)

    avail0 = jnp.where(onehot_f > 0, _NEG_INF, a0)
    mth = jnp.max(avail0, axis=1, keepdims=True)
    for _ in range(HCM_N - 2):
        cand = jnp.where(avail0 < mth, avail0, _NEG_INF)
        mth = jnp.max(cand, axis=1, keepdims=True)
    mask = jnp.maximum(onehot_f, (avail0 >= mth).astype(jnp.float32))

    ls = [x_ref[i] * SCALE + logw for i in range(c)]
    lsum = ls[0] + ls[1] + ls[2]
    pick = jnp.sum(lsum * onehot_f)

    ones_mat = jnp.ones((n, 128), jnp.float32)
    dot_dims = (((1,), (0,)), ((), ()))

    def rowsum(x):
        full = jax.lax.dot_general(x, ones_mat, dot_dims,
                                   preferred_element_type=jnp.float32)
        return full[:, :1]

    lwmax = jnp.max(logw)
    amax = jnp.max(jnp.maximum(jnp.maximum(a0, x_ref[1]), x_ref[2]),
                   axis=1, keepdims=True)
    mb = amax * SCALE + lwmax

    t_terms = []
    ce_terms = []
    for group in range(2):
        p_tot = None
        lp_tot = None
        diag = None
        ent = None
        lse_sum = None
        for i in range(c):
            if group == 0:
                v = ls[i]
                m = mb
            else:
                v = ls[i] * mask
                m = jnp.max(v, axis=1, keepdims=True)
            vm = v - m
            ex = jnp.exp(vm)
            e = jnp.sum(ex, axis=1, keepdims=True)
            loge = jnp.log(e)
            p = ex * (1.0 / e)
            logp = vm - loge
            lp = jnp.log(p + 1e-9)
            e_i = p * logp
            d_i = p * lp
            l_i = jnp.sum(m + loge)
            p_tot = p if p_tot is None else p_tot + p
            lp_tot = lp if lp_tot is None else lp_tot + lp
            ent = e_i if ent is None else ent + e_i
            diag = d_i if diag is None else diag + d_i
            lse_sum = l_i if lse_sum is None else lse_sum + l_i
        x_cross = jnp.sum(p_tot * lp_tot - diag)
        s_tot = jnp.sum(ent)
        t_terms.append((c - 1) * s_tot - x_cross)
        ce_terms.append(lse_sum - pick)

    lane = jax.lax.broadcasted_iota(jnp.int32, (1, 128), 1)
    out = jnp.where(lane == 0, t_terms[0], 0.0)
    out = jnp.where(lane == 1, t_terms[1], out)
    out = jnp.where(lane == 2, ce_terms[0], out)
    out = jnp.where(lane == 3, ce_terms[1], out)
    out_ref[...] = out[None].astype(jnp.float32)


def kernel(inputs, targets, bsce_weight):
    c, b, n = inputs.shape
    bb = 2048
    grid = b // bb
    t3 = targets[0].reshape(grid, 1, bb)
    w2 = bsce_weight.reshape(1, n)

    body = functools.partial(_block_body, bb=bb, n=n, c=c)
    partials = pl.pallas_call(
        body,
        grid=(grid,),
        in_specs=[
            pl.BlockSpec((c, bb, n), lambda g: (0, g, 0)),
            pl.BlockSpec((1, 1, bb), lambda g: (g, 0, 0)),
            pl.BlockSpec((1, n), lambda g: (0, 0)),
        ],
        out_specs=pl.BlockSpec((1, 1, 128), lambda g: (g, 0, 0)),
        out_shape=jax.ShapeDtypeStruct((grid, 1, 128), jnp.float32),
    )(inputs, t3, w2)

    s = jnp.sum(partials, axis=(0, 1))
    t_bal, t_hcm, ce_bal, ce_hcm = s[0], s[1], s[2], s[3]
    loss = (FACTOR * (t_bal + t_hcm) / (c - 1) + ce_bal + ce_hcm) / b
    return loss.astype(jnp.float32)


# speedup vs baseline: 15.9381x; 1.0097x over previous
import functools

import jax
import jax.numpy as jnp
from jax.experimental import pallas as pl

SCALE = 16.0
HCM_N = 30
FACTOR = 0.6

_NEG_INF = float("-inf")


def _block_body(x_ref, t_ref, w_ref, out_ref, *, bb, n, c):
    npad = 128
    t0 = t_ref[0, 0]
    pad_block = jnp.full((bb, npad - n), -1e5, jnp.float32)
    aps = [jnp.concatenate([x_ref[i], pad_block], axis=1) for i in range(c)]
    a0 = aps[0]
    logw = jnp.concatenate(
        [jnp.log(w_ref[...]), jnp.zeros((1, npad - n), jnp.float32)], axis=1)
    lane = jax.lax.broadcasted_iota(jnp.int32, (1, npad), 1)
    padfix = jnp.where(lane >= n, -1e5, 0.0)
    iota = jax.lax.broadcasted_iota(jnp.int32, (bb, npad), 1)
    onehot_f = (iota == t0[:, None]).astype(jnp.float32)

    avail0 = jnp.where(onehot_f > 0, _NEG_INF, a0)
    mth = jnp.max(avail0, axis=1, keepdims=True)
    for _ in range(HCM_N - 2):
        cand = jnp.where(avail0 < mth, avail0, _NEG_INF)
        mth = jnp.max(cand, axis=1, keepdims=True)
    mask = jnp.maximum(onehot_f, (avail0 >= mth).astype(jnp.float32))

    ls = [aps[i] * SCALE + logw for i in range(c)]
    lsum = ls[0] + ls[1] + ls[2]
    pick = jnp.sum(lsum * onehot_f)

    ones_mat = jnp.ones((n, 128), jnp.float32)
    dot_dims = (((1,), (0,)), ((), ()))

    def rowsum(x):
        full = jax.lax.dot_general(x, ones_mat, dot_dims,
                                   preferred_element_type=jnp.float32)
        return full[:, :1]

    lwmax = jnp.max(jnp.log(w_ref[...]))
    amax = jnp.max(jnp.maximum(jnp.maximum(a0, aps[1]), aps[2]),
                   axis=1, keepdims=True)
    mb = amax * SCALE + lwmax

    t_terms = []
    ce_terms = []
    for group in range(2):
        p_tot = None
        lp_tot = None
        diag = None
        ent = None
        lse_sum = None
        for i in range(c):
            if group == 0:
                v = ls[i]
                m = mb
            else:
                v = ls[i] * mask + padfix
                m = jnp.max(v, axis=1, keepdims=True)
            vm = v - m
            ex = jnp.exp(vm)
            e = jnp.sum(ex, axis=1, keepdims=True)
            loge = jnp.log(e)
            p = ex * (1.0 / e)
            logp = vm - loge
            lp = jnp.log(p + 1e-9)
            e_i = p * logp
            d_i = p * lp
            l_i = jnp.sum(m + loge)
            p_tot = p if p_tot is None else p_tot + p
            lp_tot = lp if lp_tot is None else lp_tot + lp
            ent = e_i if ent is None else ent + e_i
            diag = d_i if diag is None else diag + d_i
            lse_sum = l_i if lse_sum is None else lse_sum + l_i
        x_cross = jnp.sum(p_tot * lp_tot - diag)
        s_tot = jnp.sum(ent)
        t_terms.append((c - 1) * s_tot - x_cross)
        ce_terms.append(lse_sum - pick)

    out = jnp.where(lane == 0, t_terms[0], 0.0)
    out = jnp.where(lane == 1, t_terms[1], out)
    out = jnp.where(lane == 2, ce_terms[0], out)
    out = jnp.where(lane == 3, ce_terms[1], out)
    out_ref[...] = out[None].astype(jnp.float32)


def kernel(inputs, targets, bsce_weight):
    c, b, n = inputs.shape
    bb = 2048
    grid = b // bb
    t3 = targets[0].reshape(grid, 1, bb)
    w2 = bsce_weight.reshape(1, n)

    body = functools.partial(_block_body, bb=bb, n=n, c=c)
    partials = pl.pallas_call(
        body,
        grid=(grid,),
        in_specs=[
            pl.BlockSpec((c, bb, n), lambda g: (0, g, 0)),
            pl.BlockSpec((1, 1, bb), lambda g: (g, 0, 0)),
            pl.BlockSpec((1, n), lambda g: (0, 0)),
        ],
        out_specs=pl.BlockSpec((1, 1, 128), lambda g: (g, 0, 0)),
        out_shape=jax.ShapeDtypeStruct((grid, 1, 128), jnp.float32),
    )(inputs, t3, w2)

    s = jnp.sum(partials, axis=(0, 1))
    t_bal, t_hcm, ce_bal, ce_hcm = s[0], s[1], s[2], s[3]
    loss = (FACTOR * (t_bal + t_hcm) / (c - 1) + ce_bal + ce_hcm) / b
    return loss.astype(jnp.float32)


# speedup vs baseline: 15.9400x; 1.0001x over previous
import functools

import jax
import jax.numpy as jnp
from jax.experimental import pallas as pl

SCALE = 16.0
HCM_N = 30
FACTOR = 0.6

_NEG_INF = float("-inf")


def _block_body(x_ref, t_ref, w_ref, out_ref, *, bb, n, c):
    npad = 128
    t0 = t_ref[0, 0]
    pad_block = jnp.full((bb, npad - n), -1e5, jnp.float32)
    aps = [jnp.concatenate([x_ref[i], pad_block], axis=1) for i in range(c)]
    a0 = aps[0]
    logw = jnp.concatenate(
        [jnp.log(w_ref[...]), jnp.zeros((1, npad - n), jnp.float32)], axis=1)
    lane = jax.lax.broadcasted_iota(jnp.int32, (1, npad), 1)
    padfix = jnp.where(lane >= n, -1e5, 0.0)
    iota = jax.lax.broadcasted_iota(jnp.int32, (bb, npad), 1)
    onehot_f = (iota == t0[:, None]).astype(jnp.float32)

    avail0 = jnp.where(onehot_f > 0, _NEG_INF, a0)
    mth = jnp.max(avail0, axis=1, keepdims=True)
    for _ in range(HCM_N - 2):
        cand = jnp.where(avail0 < mth, avail0, _NEG_INF)
        mth = jnp.max(cand, axis=1, keepdims=True)
    mask = jnp.maximum(onehot_f, (avail0 >= mth).astype(jnp.float32))

    ls = [aps[i] * SCALE + logw for i in range(c)]
    lsum = ls[0] + ls[1] + ls[2]
    pick = jnp.sum(lsum * onehot_f)

    lwmax = jnp.max(jnp.log(w_ref[...]))
    amax = jnp.max(jnp.maximum(jnp.maximum(a0, aps[1]), aps[2]),
                   axis=1, keepdims=True)
    mb = amax * SCALE + lwmax

    t_terms = []
    ce_terms = []
    for group in range(2):
        p_tot = None
        lp_tot = None
        diag = None
        ent = None
        lse_sum = None
        for i in range(c):
            if group == 0:
                v = ls[i]
                m = mb
            else:
                v = ls[i] * mask + padfix
                m = jnp.max(v, axis=1, keepdims=True)
            vm = v - m
            ex = jnp.exp(vm)
            e = jnp.sum(ex, axis=1, keepdims=True)
            loge = jnp.log(e)
            p = ex * (1.0 / e)
            logp = vm - loge
            lp = jnp.log(p + 1e-9)
            e_i = p * logp
            d_i = p * lp
            l_i = jnp.sum(m + loge)
            p_tot = p if p_tot is None else p_tot + p
            lp_tot = lp if lp_tot is None else lp_tot + lp
            ent = e_i if ent is None else ent + e_i
            diag = d_i if diag is None else diag + d_i
            lse_sum = l_i if lse_sum is None else lse_sum + l_i
        x_cross = jnp.sum(p_tot * lp_tot - diag)
        s_tot = jnp.sum(ent)
        t_terms.append((c - 1) * s_tot - x_cross)
        ce_terms.append(lse_sum - pick)

    out = jnp.where(lane == 0, t_terms[0], 0.0)
    out = jnp.where(lane == 1, t_terms[1], out)
    out = jnp.where(lane == 2, ce_terms[0], out)
    out = jnp.where(lane == 3, ce_terms[1], out)
    out_ref[...] = out[None].astype(jnp.float32)


def kernel(inputs, targets, bsce_weight):
    c, b, n = inputs.shape
    bb = 2048
    grid = b // bb
    t3 = targets[0].reshape(grid, 1, bb)
    w2 = bsce_weight.reshape(1, n)

    body = functools.partial(_block_body, bb=bb, n=n, c=c)
    partials = pl.pallas_call(
        body,
        grid=(grid,),
        in_specs=[
            pl.BlockSpec((c, bb, n), lambda g: (0, g, 0)),
            pl.BlockSpec((1, 1, bb), lambda g: (g, 0, 0)),
            pl.BlockSpec((1, n), lambda g: (0, 0)),
        ],
        out_specs=pl.BlockSpec((1, 1, 128), lambda g: (g, 0, 0)),
        out_shape=jax.ShapeDtypeStruct((grid, 1, 128), jnp.float32),
    )(inputs, t3, w2)

    s = jnp.sum(partials, axis=(0, 1))
    t_bal, t_hcm, ce_bal, ce_hcm = s[0], s[1], s[2], s[3]
    loss = (FACTOR * (t_bal + t_hcm) / (c - 1) + ce_bal + ce_hcm) / b
    return loss.astype(jnp.float32)
